# baseline (speedup 1.0000x reference)
import functools

import jax
import jax.numpy as jnp
from jax.experimental import pallas as pl
from jax.experimental.pallas import tpu as pltpu


def _rot(x, s, size):
    s = s % size
    if s == 0:
        return x
    return jnp.concatenate([x[:, s:], x[:, :s]], axis=1)


def _fused_kernel(x_ref, wc_ref, ssel_ref, wb_ref, u2_ref, bias_ref, ba_ref,
                  m_ref, am_ref, o_ref, *, W, Wa, HW, Ma):
    cout = wb_ref.shape[0]
    xv = x_ref[0].astype(jnp.bfloat16)

    parts = []
    for t in range(9):
        kh, kw = divmod(t, 3)
        off = (kh - 1) * W + (kw - 1)
        r = _rot(xv, off, HW)
        if t != 4:
            r = r * m_ref[t]
        parts.append(r)
    patches = jnp.concatenate(parts, axis=0)

    acc9 = jnp.dot(wc_ref[...], patches, preferred_element_type=jnp.float32)
    acc = acc9[:cout]
    v = acc9[cout:cout + 1]

    a_img = jnp.dot(v.astype(jnp.bfloat16), ssel_ref[...],
                    preferred_element_type=jnp.float32) + ba_ref[...]
    a_img = a_img.astype(jnp.bfloat16)
    aparts = []
    for t in range(9):
        kh, kw = divmod(t, 3)
        off = (kh - 1) * Wa + (kw - 1)
        r = _rot(a_img, off, Ma)
        if t != 4:
            r = r * am_ref[t]
        aparts.append(r)
    a9 = jnp.concatenate(aparts, axis=0)

    ls = jnp.dot(wb_ref[...], a9, preferred_element_type=jnp.float32)
    up = jnp.dot(ls.astype(jnp.bfloat16), u2_ref[...],
                 preferred_element_type=jnp.float32)

    o_ref[0] = (acc + up + bias_ref[...]).astype(o_ref.dtype)


def kernel(x, w_fixed, b_fixed, w_a, b_a, w_b, b_b):
    N, Cin, H, W = x.shape
    Cout = w_fixed.shape[0]
    HW = H * W
    Ha, Wa = H // 4, W // 4
    Ma = Ha * Wa
    dtype = x.dtype
    bf = jnp.bfloat16

    xf = x.reshape(N, Cin, HW)

    wc = jnp.concatenate([
        jnp.transpose(w_fixed, (0, 2, 3, 1)).reshape(Cout, Cin * 9),
        jnp.transpose(w_a, (0, 2, 3, 1)).reshape(1, Cin * 9),
    ], axis=0).astype(bf)
    wb9 = w_b.reshape(Cout, 9).astype(bf)
    bias = (b_fixed + b_b).reshape(Cout, 1)
    ba = b_a.reshape(1, 1)

    hh = jnp.arange(HW) // W
    ww = jnp.arange(HW) % W
    ha = jnp.arange(Ma) // Wa
    wa_ = jnp.arange(Ma) % Wa
    masks, amasks = [], []
    for t in range(9):
        kh, kw = divmod(t, 3)
        masks.append(((hh + kh - 1 >= 0) & (hh + kh - 1 < H)
                      & (ww + kw - 1 >= 0) & (ww + kw - 1 < W)))
        amasks.append(((ha + kh - 1 >= 0) & (ha + kh - 1 < Ha)
                       & (wa_ + kw - 1 >= 0) & (wa_ + kw - 1 < Wa)))
    m9 = jnp.stack(masks).reshape(9, 1, HW).astype(bf)
    am9 = jnp.stack(amasks).reshape(9, 1, Ma).astype(bf)

    q_of_m = (ha * 4) * W + wa_ * 4
    ssel = (jnp.arange(HW)[:, None] == q_of_m[None, :]).astype(bf)
    m_of_q = (hh // 4) * Wa + ww // 4
    u2 = (jnp.arange(Ma)[:, None] == m_of_q[None, :]).astype(bf)

    flops = int(N * (2 * (Cout + 1) * Cin * 9 * HW + 2 * HW * Ma
                     + 2 * Cout * 9 * Ma + 2 * Cout * Ma * HW))
    bytes_accessed = int(4 * (N * Cin * HW + N * Cout * HW)
                         + 2 * (HW * Ma * 2 + 9 * HW + 9 * Ma))

    kern = functools.partial(_fused_kernel, W=W, Wa=Wa, HW=HW, Ma=Ma)
    out = pl.pallas_call(
        kern,
        out_shape=jax.ShapeDtypeStruct((N, Cout, HW), dtype),
        grid=(N,),
        in_specs=[
            pl.BlockSpec((1, Cin, HW), lambda n: (n, 0, 0)),
            pl.BlockSpec((Cout + 1, Cin * 9), lambda n: (0, 0)),
            pl.BlockSpec((HW, Ma), lambda n: (0, 0)),
            pl.BlockSpec((Cout, 9), lambda n: (0, 0)),
            pl.BlockSpec((Ma, HW), lambda n: (0, 0)),
            pl.BlockSpec((Cout, 1), lambda n: (0, 0)),
            pl.BlockSpec((1, 1), lambda n: (0, 0)),
            pl.BlockSpec((9, 1, HW), lambda n: (0, 0, 0)),
            pl.BlockSpec((9, 1, Ma), lambda n: (0, 0, 0)),
        ],
        out_specs=pl.BlockSpec((1, Cout, HW), lambda n: (n, 0, 0)),
        compiler_params=pltpu.CompilerParams(dimension_semantics=("parallel",)),
        cost_estimate=pl.CostEstimate(flops=flops, transcendentals=0,
                                      bytes_accessed=bytes_accessed),
    )(xf, wc, ssel, wb9, u2, bias, ba, m9, am9)

    return out.reshape(N, Cout, H, W)


# speedup vs baseline: 1.0007x; 1.0007x over previous
import functools

import jax
import jax.numpy as jnp
from jax.experimental import pallas as pl
from jax.experimental.pallas import tpu as pltpu


def _fused_kernel(x_ref, wc_ref, ssel_ref, wb_ref, u2_ref, bias_ref, ba_ref,
                  m_ref, am_ref, o_ref, *, W, Wa, HW, Ma):
    cout = wb_ref.shape[0]
    xv = x_ref[0]

    parts = []
    for t in range(9):
        kh, kw = divmod(t, 3)
        off = (kh - 1) * W + (kw - 1)
        r = pltpu.roll(xv, (-off) % HW, axis=1) if off != 0 else xv
        if t != 4:
            r = r * m_ref[t]
        parts.append(r)
    patches = jnp.concatenate(parts, axis=0).astype(jnp.bfloat16)

    acc9 = jnp.dot(wc_ref[...], patches, preferred_element_type=jnp.float32)
    acc = acc9[:cout]
    v = acc9[cout:cout + 1]

    a_img = jnp.dot(v.astype(jnp.bfloat16), ssel_ref[...],
                    preferred_element_type=jnp.float32) + ba_ref[...]
    aparts = []
    for t in range(9):
        kh, kw = divmod(t, 3)
        off = (kh - 1) * Wa + (kw - 1)
        r = pltpu.roll(a_img, (-off) % Ma, axis=1) if off != 0 else a_img
        if t != 4:
            r = r * am_ref[t]
        aparts.append(r)
    a9 = jnp.concatenate(aparts, axis=0).astype(jnp.bfloat16)

    ls = jnp.dot(wb_ref[...], a9, preferred_element_type=jnp.float32)
    up = jnp.dot(ls.astype(jnp.bfloat16), u2_ref[...],
                 preferred_element_type=jnp.float32)

    o_ref[0] = (acc + up + bias_ref[...]).astype(o_ref.dtype)


def kernel(x, w_fixed, b_fixed, w_a, b_a, w_b, b_b):
    N, Cin, H, W = x.shape
    Cout = w_fixed.shape[0]
    HW = H * W
    Ha, Wa = H // 4, W // 4
    Ma = Ha * Wa
    dtype = x.dtype
    bf = jnp.bfloat16

    xf = x.reshape(N, Cin, HW)

    wc = jnp.concatenate([
        jnp.transpose(w_fixed, (0, 2, 3, 1)).reshape(Cout, Cin * 9),
        jnp.transpose(w_a, (0, 2, 3, 1)).reshape(1, Cin * 9),
    ], axis=0).astype(bf)
    wb9 = w_b.reshape(Cout, 9).astype(bf)
    bias = (b_fixed + b_b).reshape(Cout, 1)
    ba = b_a.reshape(1, 1)

    hh = jnp.arange(HW) // W
    ww = jnp.arange(HW) % W
    ha = jnp.arange(Ma) // Wa
    wa_ = jnp.arange(Ma) % Wa
    masks, amasks = [], []
    for t in range(9):
        kh, kw = divmod(t, 3)
        masks.append(((hh + kh - 1 >= 0) & (hh + kh - 1 < H)
                      & (ww + kw - 1 >= 0) & (ww + kw - 1 < W)))
        amasks.append(((ha + kh - 1 >= 0) & (ha + kh - 1 < Ha)
                       & (wa_ + kw - 1 >= 0) & (wa_ + kw - 1 < Wa)))
    m9 = jnp.stack(masks).reshape(9, 1, HW).astype(jnp.float32)
    am9 = jnp.stack(amasks).reshape(9, 1, Ma).astype(jnp.float32)

    q_of_m = (ha * 4) * W + wa_ * 4
    ssel = (jnp.arange(HW)[:, None] == q_of_m[None, :]).astype(bf)
    m_of_q = (hh // 4) * Wa + ww // 4
    u2 = (jnp.arange(Ma)[:, None] == m_of_q[None, :]).astype(bf)

    flops = int(N * (2 * (Cout + 1) * Cin * 9 * HW + 2 * HW * Ma
                     + 2 * Cout * 9 * Ma + 2 * Cout * Ma * HW))
    bytes_accessed = int(4 * (N * Cin * HW + N * Cout * HW)
                         + 2 * (HW * Ma * 2 + 9 * HW + 9 * Ma))

    kern = functools.partial(_fused_kernel, W=W, Wa=Wa, HW=HW, Ma=Ma)
    out = pl.pallas_call(
        kern,
        out_shape=jax.ShapeDtypeStruct((N, Cout, HW), dtype),
        grid=(N,),
        in_specs=[
            pl.BlockSpec((1, Cin, HW), lambda n: (n, 0, 0)),
            pl.BlockSpec((Cout + 1, Cin * 9), lambda n: (0, 0)),
            pl.BlockSpec((HW, Ma), lambda n: (0, 0)),
            pl.BlockSpec((Cout, 9), lambda n: (0, 0)),
            pl.BlockSpec((Ma, HW), lambda n: (0, 0)),
            pl.BlockSpec((Cout, 1), lambda n: (0, 0)),
            pl.BlockSpec((1, 1), lambda n: (0, 0)),
            pl.BlockSpec((9, 1, HW), lambda n: (0, 0, 0)),
            pl.BlockSpec((9, 1, Ma), lambda n: (0, 0, 0)),
        ],
        out_specs=pl.BlockSpec((1, Cout, HW), lambda n: (n, 0, 0)),
        compiler_params=pltpu.CompilerParams(dimension_semantics=("parallel",)),
        cost_estimate=pl.CostEstimate(flops=flops, transcendentals=0,
                                      bytes_accessed=bytes_accessed),
    )(xf, wc, ssel, wb9, u2, bias, ba, m9, am9)

    return out.reshape(N, Cout, H, W)


# speedup vs baseline: 1.1258x; 1.1250x over previous
import functools

import jax
import jax.numpy as jnp
from jax.experimental import pallas as pl
from jax.experimental.pallas import tpu as pltpu


def _fused_kernel(x_ref, wc_ref, ssel_ref, wb_ref, u2_ref, bias_ref, ba_ref,
                  m_ref, am_ref, o_ref, *, W, Wa, HW, Ma, B):
    cout = wb_ref.shape[0]
    for b in range(B):
        xv = x_ref[b]

        parts = []
        for t in range(9):
            kh, kw = divmod(t, 3)
            off = (kh - 1) * W + (kw - 1)
            r = pltpu.roll(xv, (-off) % HW, axis=1) if off != 0 else xv
            if t != 4:
                r = r * m_ref[t]
            parts.append(r)
        patches = jnp.concatenate(parts, axis=0).astype(jnp.bfloat16)

        acc9 = jnp.dot(wc_ref[...], patches, preferred_element_type=jnp.float32)
        acc = acc9[:cout]
        v = acc9[cout:cout + 1]

        a_img = jnp.dot(v.astype(jnp.bfloat16), ssel_ref[...],
                        preferred_element_type=jnp.float32) + ba_ref[...]
        aparts = []
        for t in range(9):
            kh, kw = divmod(t, 3)
            off = (kh - 1) * Wa + (kw - 1)
            r = pltpu.roll(a_img, (-off) % Ma, axis=1) if off != 0 else a_img
            if t != 4:
                r = r * am_ref[t]
            aparts.append(r)
        a9 = jnp.concatenate(aparts, axis=0).astype(jnp.bfloat16)

        ls = jnp.dot(wb_ref[...], a9, preferred_element_type=jnp.float32)
        up = jnp.dot(ls.astype(jnp.bfloat16), u2_ref[...],
                     preferred_element_type=jnp.float32)

        o_ref[b] = (acc + up + bias_ref[...]).astype(o_ref.dtype)


def kernel(x, w_fixed, b_fixed, w_a, b_a, w_b, b_b):
    N, Cin, H, W = x.shape
    Cout = w_fixed.shape[0]
    HW = H * W
    Ha, Wa = H // 4, W // 4
    Ma = Ha * Wa
    dtype = x.dtype
    bf = jnp.bfloat16

    xf = x.reshape(N, Cin, HW)

    wc = jnp.concatenate([
        jnp.transpose(w_fixed, (0, 2, 3, 1)).reshape(Cout, Cin * 9),
        jnp.transpose(w_a, (0, 2, 3, 1)).reshape(1, Cin * 9),
    ], axis=0).astype(bf)
    wb9 = w_b.reshape(Cout, 9).astype(bf)
    bias = (b_fixed + b_b).reshape(Cout, 1)
    ba = b_a.reshape(1, 1)

    hh = jnp.arange(HW) // W
    ww = jnp.arange(HW) % W
    ha = jnp.arange(Ma) // Wa
    wa_ = jnp.arange(Ma) % Wa
    masks, amasks = [], []
    for t in range(9):
        kh, kw = divmod(t, 3)
        masks.append(((hh + kh - 1 >= 0) & (hh + kh - 1 < H)
                      & (ww + kw - 1 >= 0) & (ww + kw - 1 < W)))
        amasks.append(((ha + kh - 1 >= 0) & (ha + kh - 1 < Ha)
                       & (wa_ + kw - 1 >= 0) & (wa_ + kw - 1 < Wa)))
    m9 = jnp.stack(masks).reshape(9, 1, HW).astype(jnp.float32)
    am9 = jnp.stack(amasks).reshape(9, 1, Ma).astype(jnp.float32)

    q_of_m = (ha * 4) * W + wa_ * 4
    ssel = (jnp.arange(HW)[:, None] == q_of_m[None, :]).astype(bf)
    m_of_q = (hh // 4) * Wa + ww // 4
    u2 = (jnp.arange(Ma)[:, None] == m_of_q[None, :]).astype(bf)

    flops = int(N * (2 * (Cout + 1) * Cin * 9 * HW + 2 * HW * Ma
                     + 2 * Cout * 9 * Ma + 2 * Cout * Ma * HW))
    bytes_accessed = int(4 * (N * Cin * HW + N * Cout * HW)
                         + 2 * (HW * Ma * 2 + 9 * HW + 9 * Ma))

    B = 4
    while N % B:
        B //= 2
    kern = functools.partial(_fused_kernel, W=W, Wa=Wa, HW=HW, Ma=Ma, B=B)
    out = pl.pallas_call(
        kern,
        out_shape=jax.ShapeDtypeStruct((N, Cout, HW), dtype),
        grid=(N // B,),
        in_specs=[
            pl.BlockSpec((B, Cin, HW), lambda n: (n, 0, 0)),
            pl.BlockSpec((Cout + 1, Cin * 9), lambda n: (0, 0)),
            pl.BlockSpec((HW, Ma), lambda n: (0, 0)),
            pl.BlockSpec((Cout, 9), lambda n: (0, 0)),
            pl.BlockSpec((Ma, HW), lambda n: (0, 0)),
            pl.BlockSpec((Cout, 1), lambda n: (0, 0)),
            pl.BlockSpec((1, 1), lambda n: (0, 0)),
            pl.BlockSpec((9, 1, HW), lambda n: (0, 0, 0)),
            pl.BlockSpec((9, 1, Ma), lambda n: (0, 0, 0)),
        ],
        out_specs=pl.BlockSpec((B, Cout, HW), lambda n: (n, 0, 0)),
        compiler_params=pltpu.CompilerParams(dimension_semantics=("parallel",)),
        cost_estimate=pl.CostEstimate(flops=flops, transcendentals=0,
                                      bytes_accessed=bytes_accessed),
    )(xf, wc, ssel, wb9, u2, bias, ba, m9, am9)

    return out.reshape(N, Cout, H, W)


# speedup vs baseline: 1.5499x; 1.3768x over previous
import functools

import jax
import jax.numpy as jnp
from jax.experimental import pallas as pl
from jax.experimental.pallas import tpu as pltpu


def _fused_kernel(x_ref, wc_ref, ssel_ref, wb_ref, u2_ref, bias_ref, ba_ref,
                  m_ref, am_ref, o_ref, *, W, Wa, HW, Ma, B):
    cout = wb_ref.shape[0]
    for b in range(B):
        xv = x_ref[b]

        parts = []
        for t in range(9):
            kh, kw = divmod(t, 3)
            off = (kh - 1) * W + (kw - 1)
            r = pltpu.roll(xv, (-off) % HW, axis=1) if off != 0 else xv
            if t != 4:
                r = r * m_ref[t]
            parts.append(r)
        patches = jnp.concatenate(parts, axis=0)

        acc9 = jnp.dot(wc_ref[...], patches, preferred_element_type=jnp.float32)
        acc = acc9[:cout]
        v = acc9[cout:cout + 1]

        a_img = jnp.dot(v, ssel_ref[...],
                        preferred_element_type=jnp.float32) + ba_ref[...]
        aparts = []
        for t in range(9):
            kh, kw = divmod(t, 3)
            off = (kh - 1) * Wa + (kw - 1)
            r = pltpu.roll(a_img, (-off) % Ma, axis=1) if off != 0 else a_img
            if t != 4:
                r = r * am_ref[t]
            aparts.append(r)
        a9 = jnp.concatenate(aparts, axis=0)

        ls = jnp.dot(wb_ref[...], a9, preferred_element_type=jnp.float32)
        up = jnp.dot(ls, u2_ref[...],
                     preferred_element_type=jnp.float32)

        o_ref[b] = (acc + up + bias_ref[...]).astype(o_ref.dtype)


def kernel(x, w_fixed, b_fixed, w_a, b_a, w_b, b_b):
    N, Cin, H, W = x.shape
    Cout = w_fixed.shape[0]
    HW = H * W
    Ha, Wa = H // 4, W // 4
    Ma = Ha * Wa
    dtype = x.dtype
    bf = jnp.bfloat16

    xf = x.reshape(N, Cin, HW)

    wc = jnp.concatenate([
        jnp.transpose(w_fixed, (0, 2, 3, 1)).reshape(Cout, Cin * 9),
        jnp.transpose(w_a, (0, 2, 3, 1)).reshape(1, Cin * 9),
    ], axis=0)
    wb9 = w_b.reshape(Cout, 9)
    bias = (b_fixed + b_b).reshape(Cout, 1)
    ba = b_a.reshape(1, 1)

    hh = jnp.arange(HW) // W
    ww = jnp.arange(HW) % W
    ha = jnp.arange(Ma) // Wa
    wa_ = jnp.arange(Ma) % Wa
    masks, amasks = [], []
    for t in range(9):
        kh, kw = divmod(t, 3)
        masks.append(((hh + kh - 1 >= 0) & (hh + kh - 1 < H)
                      & (ww + kw - 1 >= 0) & (ww + kw - 1 < W)))
        amasks.append(((ha + kh - 1 >= 0) & (ha + kh - 1 < Ha)
                       & (wa_ + kw - 1 >= 0) & (wa_ + kw - 1 < Wa)))
    m9 = jnp.stack(masks).reshape(9, 1, HW).astype(jnp.float32)
    am9 = jnp.stack(amasks).reshape(9, 1, Ma).astype(jnp.float32)

    q_of_m = (ha * 4) * W + wa_ * 4
    ssel = (jnp.arange(HW)[:, None] == q_of_m[None, :]).astype(jnp.float32)
    m_of_q = (hh // 4) * Wa + ww // 4
    u2 = (jnp.arange(Ma)[:, None] == m_of_q[None, :]).astype(jnp.float32)

    flops = int(N * (2 * (Cout + 1) * Cin * 9 * HW + 2 * HW * Ma
                     + 2 * Cout * 9 * Ma + 2 * Cout * Ma * HW))
    bytes_accessed = int(4 * (N * Cin * HW + N * Cout * HW)
                         + 2 * (HW * Ma * 2 + 9 * HW + 9 * Ma))

    B = 4
    while N % B:
        B //= 2
    kern = functools.partial(_fused_kernel, W=W, Wa=Wa, HW=HW, Ma=Ma, B=B)
    out = pl.pallas_call(
        kern,
        out_shape=jax.ShapeDtypeStruct((N, Cout, HW), dtype),
        grid=(N // B,),
        in_specs=[
            pl.BlockSpec((B, Cin, HW), lambda n: (n, 0, 0)),
            pl.BlockSpec((Cout + 1, Cin * 9), lambda n: (0, 0)),
            pl.BlockSpec((HW, Ma), lambda n: (0, 0)),
            pl.BlockSpec((Cout, 9), lambda n: (0, 0)),
            pl.BlockSpec((Ma, HW), lambda n: (0, 0)),
            pl.BlockSpec((Cout, 1), lambda n: (0, 0)),
            pl.BlockSpec((1, 1), lambda n: (0, 0)),
            pl.BlockSpec((9, 1, HW), lambda n: (0, 0, 0)),
            pl.BlockSpec((9, 1, Ma), lambda n: (0, 0, 0)),
        ],
        out_specs=pl.BlockSpec((B, Cout, HW), lambda n: (n, 0, 0)),
        compiler_params=pltpu.CompilerParams(dimension_semantics=("parallel",)),
        cost_estimate=pl.CostEstimate(flops=flops, transcendentals=0,
                                      bytes_accessed=bytes_accessed),
    )(xf, wc, ssel, wb9, u2, bias, ba, m9, am9)

    return out.reshape(N, Cout, H, W)


# speedup vs baseline: 2.7490x; 1.7737x over previous
import functools

import jax
import jax.numpy as jnp
from jax.experimental import pallas as pl
from jax.experimental.pallas import tpu as pltpu


def _fused_kernel(x_ref, wc_ref, ssel_ref, wb2_ref, u2_ref, bias_ref, ba_ref,
                  m_ref, am_ref, o_ref, *, W, Wa, HW, Ma, B, Cout):
    parts_all = []
    for b in range(B):
        xv = x_ref[b]
        parts = []
        for t in range(9):
            kh, kw = divmod(t, 3)
            off = (kh - 1) * W + (kw - 1)
            r = pltpu.roll(xv, (-off) % HW, axis=1) if off != 0 else xv
            if t != 4:
                r = r * m_ref[t]
            parts.append(r)
        parts_all.append(jnp.concatenate(parts, axis=0))
    p_all = jnp.concatenate(parts_all, axis=1)

    acc9 = jnp.dot(wc_ref[...], p_all, preferred_element_type=jnp.float32)

    v_all = jnp.concatenate(
        [acc9[Cout:Cout + 1, b * HW:(b + 1) * HW] for b in range(B)],
        axis=0)
    a_all = jnp.dot(v_all, ssel_ref[...],
                    preferred_element_type=jnp.float32) + ba_ref[...]

    aparts = []
    for t in range(9):
        kh, kw = divmod(t, 3)
        off = (kh - 1) * Wa + (kw - 1)
        r = pltpu.roll(a_all, (-off) % Ma, axis=1) if off != 0 else a_all
        if t != 4:
            r = r * am_ref[t]
        aparts.append(r)
    a9 = jnp.concatenate(aparts, axis=0)

    ls = jnp.dot(wb2_ref[...], a9, preferred_element_type=jnp.float32)
    up = jnp.dot(ls, u2_ref[...],
                 preferred_element_type=jnp.float32)

    for b in range(B):
        o_ref[b] = (acc9[:Cout, b * HW:(b + 1) * HW]
                    + up[b * Cout:(b + 1) * Cout]
                    + bias_ref[...]).astype(o_ref.dtype)


def kernel(x, w_fixed, b_fixed, w_a, b_a, w_b, b_b):
    N, Cin, H, W = x.shape
    Cout = w_fixed.shape[0]
    HW = H * W
    Ha, Wa = H // 4, W // 4
    Ma = Ha * Wa
    dtype = x.dtype
    bf = jnp.bfloat16

    xf = x.reshape(N, Cin, HW)

    wc = jnp.concatenate([
        jnp.transpose(w_fixed, (0, 2, 3, 1)).reshape(Cout, Cin * 9),
        jnp.transpose(w_a, (0, 2, 3, 1)).reshape(1, Cin * 9),
    ], axis=0)
    wb9 = w_b.reshape(Cout, 9)
    bias = (b_fixed + b_b).reshape(Cout, 1)
    ba = b_a.reshape(1, 1)

    hh = jnp.arange(HW) // W
    ww = jnp.arange(HW) % W
    ha = jnp.arange(Ma) // Wa
    wa_ = jnp.arange(Ma) % Wa
    masks, amasks = [], []
    for t in range(9):
        kh, kw = divmod(t, 3)
        masks.append(((hh + kh - 1 >= 0) & (hh + kh - 1 < H)
                      & (ww + kw - 1 >= 0) & (ww + kw - 1 < W)))
        amasks.append(((ha + kh - 1 >= 0) & (ha + kh - 1 < Ha)
                       & (wa_ + kw - 1 >= 0) & (wa_ + kw - 1 < Wa)))
    m9 = jnp.stack(masks).reshape(9, 1, HW).astype(jnp.float32)
    am9 = jnp.stack(amasks).reshape(9, 1, Ma).astype(jnp.float32)

    q_of_m = (ha * 4) * W + wa_ * 4
    ssel = (jnp.arange(HW)[:, None] == q_of_m[None, :]).astype(jnp.float32)
    m_of_q = (hh // 4) * Wa + ww // 4
    u2 = (jnp.arange(Ma)[:, None] == m_of_q[None, :]).astype(jnp.float32)

    flops = int(N * (2 * (Cout + 1) * Cin * 9 * HW + 2 * HW * Ma
                     + 2 * Cout * 9 * Ma + 2 * Cout * Ma * HW))
    bytes_accessed = int(4 * (N * Cin * HW + N * Cout * HW)
                         + 2 * (HW * Ma * 2 + 9 * HW + 9 * Ma))

    B = 4
    while N % B:
        B //= 2

    eyeb = jnp.eye(B, dtype=jnp.float32)
    wb2 = (wb9[None, :, :, None] * eyeb[:, None, None, :]).reshape(
        B * Cout, 9 * B)

    kern = functools.partial(_fused_kernel, W=W, Wa=Wa, HW=HW, Ma=Ma, B=B,
                             Cout=Cout)
    out = pl.pallas_call(
        kern,
        out_shape=jax.ShapeDtypeStruct((N, Cout, HW), dtype),
        grid=(N // B,),
        in_specs=[
            pl.BlockSpec((B, Cin, HW), lambda n: (n, 0, 0)),
            pl.BlockSpec((Cout + 1, Cin * 9), lambda n: (0, 0)),
            pl.BlockSpec((HW, Ma), lambda n: (0, 0)),
            pl.BlockSpec((B * Cout, 9 * B), lambda n: (0, 0)),
            pl.BlockSpec((Ma, HW), lambda n: (0, 0)),
            pl.BlockSpec((Cout, 1), lambda n: (0, 0)),
            pl.BlockSpec((1, 1), lambda n: (0, 0)),
            pl.BlockSpec((9, 1, HW), lambda n: (0, 0, 0)),
            pl.BlockSpec((9, 1, Ma), lambda n: (0, 0, 0)),
        ],
        out_specs=pl.BlockSpec((B, Cout, HW), lambda n: (n, 0, 0)),
        compiler_params=pltpu.CompilerParams(dimension_semantics=("parallel",)),
        cost_estimate=pl.CostEstimate(flops=flops, transcendentals=0,
                                      bytes_accessed=bytes_accessed),
    )(xf, wc, ssel, wb2, u2, bias, ba, m9, am9)

    return out.reshape(N, Cout, H, W)


# speedup vs baseline: 3.3899x; 1.2331x over previous
import functools

import jax
import jax.numpy as jnp
from jax.experimental import pallas as pl
from jax.experimental.pallas import tpu as pltpu


def _fused_kernel(x_ref, wc_ref, ssel_ref, wb2_ref, u2_ref, bias_ref, ba_ref,
                  m_ref, am_ref, o_ref, *, W, Wa, HW, Ma, B, Cout):
    parts_all = []
    for b in range(B):
        xv = x_ref[b]
        parts = []
        for t in range(9):
            kh, kw = divmod(t, 3)
            off = (kh - 1) * W + (kw - 1)
            r = pltpu.roll(xv, (-off) % HW, axis=1) if off != 0 else xv
            if t != 4:
                r = r * m_ref[t]
            parts.append(r)
        parts_all.append(jnp.concatenate(parts, axis=0))
    p_all = jnp.concatenate(parts_all, axis=1)

    acc9 = jnp.dot(wc_ref[...], p_all, preferred_element_type=jnp.float32)

    v_all = jnp.concatenate(
        [acc9[Cout:Cout + 1, b * HW:(b + 1) * HW] for b in range(B)],
        axis=0)
    a_all = jnp.dot(v_all, ssel_ref[...],
                    preferred_element_type=jnp.float32) + ba_ref[...]

    aparts = []
    for t in range(9):
        kh, kw = divmod(t, 3)
        off = (kh - 1) * Wa + (kw - 1)
        r = pltpu.roll(a_all, (-off) % Ma, axis=1) if off != 0 else a_all
        if t != 4:
            r = r * am_ref[t]
        aparts.append(r)
    a9 = jnp.concatenate(aparts, axis=0)

    ls = jnp.dot(wb2_ref[...], a9, preferred_element_type=jnp.float32)
    up = jnp.dot(ls, u2_ref[...],
                 preferred_element_type=jnp.float32)

    for b in range(B):
        o_ref[b] = (acc9[:Cout, b * HW:(b + 1) * HW]
                    + up[b * Cout:(b + 1) * Cout]
                    + bias_ref[...]).astype(o_ref.dtype)


def kernel(x, w_fixed, b_fixed, w_a, b_a, w_b, b_b):
    N, Cin, H, W = x.shape
    Cout = w_fixed.shape[0]
    HW = H * W
    Ha, Wa = H // 4, W // 4
    Ma = Ha * Wa
    dtype = x.dtype
    bf = jnp.bfloat16

    xf = x.reshape(N, Cin, HW)

    wc = jnp.concatenate([
        jnp.transpose(w_fixed, (0, 2, 3, 1)).reshape(Cout, Cin * 9),
        jnp.transpose(w_a, (0, 2, 3, 1)).reshape(1, Cin * 9),
    ], axis=0)
    wb9 = w_b.reshape(Cout, 9)
    bias = (b_fixed + b_b).reshape(Cout, 1)
    ba = b_a.reshape(1, 1)

    hh = jnp.arange(HW) // W
    ww = jnp.arange(HW) % W
    ha = jnp.arange(Ma) // Wa
    wa_ = jnp.arange(Ma) % Wa
    masks, amasks = [], []
    for t in range(9):
        kh, kw = divmod(t, 3)
        masks.append(((hh + kh - 1 >= 0) & (hh + kh - 1 < H)
                      & (ww + kw - 1 >= 0) & (ww + kw - 1 < W)))
        amasks.append(((ha + kh - 1 >= 0) & (ha + kh - 1 < Ha)
                       & (wa_ + kw - 1 >= 0) & (wa_ + kw - 1 < Wa)))
    m9 = jnp.stack(masks).reshape(9, 1, HW).astype(jnp.float32)
    am9 = jnp.stack(amasks).reshape(9, 1, Ma).astype(jnp.float32)

    q_of_m = (ha * 4) * W + wa_ * 4
    ssel = (jnp.arange(HW)[:, None] == q_of_m[None, :]).astype(jnp.float32)
    m_of_q = (hh // 4) * Wa + ww // 4
    u2 = (jnp.arange(Ma)[:, None] == m_of_q[None, :]).astype(jnp.float32)

    flops = int(N * (2 * (Cout + 1) * Cin * 9 * HW + 2 * HW * Ma
                     + 2 * Cout * 9 * Ma + 2 * Cout * Ma * HW))
    bytes_accessed = int(4 * (N * Cin * HW + N * Cout * HW)
                         + 2 * (HW * Ma * 2 + 9 * HW + 9 * Ma))

    B = 8
    while N % B:
        B //= 2

    eyeb = jnp.eye(B, dtype=jnp.float32)
    wb2 = (wb9[None, :, :, None] * eyeb[:, None, None, :]).reshape(
        B * Cout, 9 * B)

    kern = functools.partial(_fused_kernel, W=W, Wa=Wa, HW=HW, Ma=Ma, B=B,
                             Cout=Cout)
    out = pl.pallas_call(
        kern,
        out_shape=jax.ShapeDtypeStruct((N, Cout, HW), dtype),
        grid=(N // B,),
        in_specs=[
            pl.BlockSpec((B, Cin, HW), lambda n: (n, 0, 0)),
            pl.BlockSpec((Cout + 1, Cin * 9), lambda n: (0, 0)),
            pl.BlockSpec((HW, Ma), lambda n: (0, 0)),
            pl.BlockSpec((B * Cout, 9 * B), lambda n: (0, 0)),
            pl.BlockSpec((Ma, HW), lambda n: (0, 0)),
            pl.BlockSpec((Cout, 1), lambda n: (0, 0)),
            pl.BlockSpec((1, 1), lambda n: (0, 0)),
            pl.BlockSpec((9, 1, HW), lambda n: (0, 0, 0)),
            pl.BlockSpec((9, 1, Ma), lambda n: (0, 0, 0)),
        ],
        out_specs=pl.BlockSpec((B, Cout, HW), lambda n: (n, 0, 0)),
        compiler_params=pltpu.CompilerParams(dimension_semantics=("parallel",)),
        cost_estimate=pl.CostEstimate(flops=flops, transcendentals=0,
                                      bytes_accessed=bytes_accessed),
    )(xf, wc, ssel, wb2, u2, bias, ba, m9, am9)

    return out.reshape(N, Cout, H, W)


# speedup vs baseline: 3.7401x; 1.1033x over previous
import functools

import jax
import jax.numpy as jnp
from jax.experimental import pallas as pl
from jax.experimental.pallas import tpu as pltpu


def _fused_kernel(x_ref, wc_ref, ssel_ref, wb2_ref, u2_ref, bias_ref, ba_ref,
                  m_ref, am_ref, o_ref, *, W, Wa, HW, Ma, B, Cout):
    parts_all = []
    for b in range(B):
        xv = x_ref[b]
        parts = []
        for t in range(9):
            kh, kw = divmod(t, 3)
            off = (kh - 1) * W + (kw - 1)
            r = pltpu.roll(xv, (-off) % HW, axis=1) if off != 0 else xv
            if t != 4:
                r = r * m_ref[t]
            parts.append(r)
        parts_all.append(jnp.concatenate(parts, axis=0))
    p_all = jnp.concatenate(parts_all, axis=1)

    acc9 = jnp.dot(wc_ref[...], p_all, preferred_element_type=jnp.float32)

    v_all = jnp.concatenate(
        [acc9[Cout:Cout + 1, b * HW:(b + 1) * HW] for b in range(B)],
        axis=0)
    a_all = jnp.dot(v_all, ssel_ref[...],
                    preferred_element_type=jnp.float32) + ba_ref[...]

    aparts = []
    for t in range(9):
        kh, kw = divmod(t, 3)
        off = (kh - 1) * Wa + (kw - 1)
        r = pltpu.roll(a_all, (-off) % Ma, axis=1) if off != 0 else a_all
        if t != 4:
            r = r * am_ref[t]
        aparts.append(r)
    a9 = jnp.concatenate(aparts, axis=0)

    ls = jnp.dot(wb2_ref[...], a9, preferred_element_type=jnp.float32)
    up = jnp.dot(ls, u2_ref[...],
                 preferred_element_type=jnp.float32)

    for b in range(B):
        o_ref[b] = (acc9[:Cout, b * HW:(b + 1) * HW]
                    + up[b * Cout:(b + 1) * Cout]
                    + bias_ref[...]).astype(o_ref.dtype)


def kernel(x, w_fixed, b_fixed, w_a, b_a, w_b, b_b):
    N, Cin, H, W = x.shape
    Cout = w_fixed.shape[0]
    HW = H * W
    Ha, Wa = H // 4, W // 4
    Ma = Ha * Wa
    dtype = x.dtype
    bf = jnp.bfloat16

    xf = x.reshape(N, Cin, HW)

    wc = jnp.concatenate([
        jnp.transpose(w_fixed, (0, 2, 3, 1)).reshape(Cout, Cin * 9),
        jnp.transpose(w_a, (0, 2, 3, 1)).reshape(1, Cin * 9),
    ], axis=0)
    wb9 = w_b.reshape(Cout, 9)
    bias = (b_fixed + b_b).reshape(Cout, 1)
    ba = b_a.reshape(1, 1)

    hh = jnp.arange(HW) // W
    ww = jnp.arange(HW) % W
    ha = jnp.arange(Ma) // Wa
    wa_ = jnp.arange(Ma) % Wa
    masks, amasks = [], []
    for t in range(9):
        kh, kw = divmod(t, 3)
        masks.append(((hh + kh - 1 >= 0) & (hh + kh - 1 < H)
                      & (ww + kw - 1 >= 0) & (ww + kw - 1 < W)))
        amasks.append(((ha + kh - 1 >= 0) & (ha + kh - 1 < Ha)
                       & (wa_ + kw - 1 >= 0) & (wa_ + kw - 1 < Wa)))
    m9 = jnp.stack(masks).reshape(9, 1, HW).astype(jnp.float32)
    am9 = jnp.stack(amasks).reshape(9, 1, Ma).astype(jnp.float32)

    q_of_m = (ha * 4) * W + wa_ * 4
    ssel = (jnp.arange(HW)[:, None] == q_of_m[None, :]).astype(jnp.float32)
    m_of_q = (hh // 4) * Wa + ww // 4
    u2 = (jnp.arange(Ma)[:, None] == m_of_q[None, :]).astype(jnp.float32)

    flops = int(N * (2 * (Cout + 1) * Cin * 9 * HW + 2 * HW * Ma
                     + 2 * Cout * 9 * Ma + 2 * Cout * Ma * HW))
    bytes_accessed = int(4 * (N * Cin * HW + N * Cout * HW)
                         + 2 * (HW * Ma * 2 + 9 * HW + 9 * Ma))

    B = 16
    while N % B:
        B //= 2

    eyeb = jnp.eye(B, dtype=jnp.float32)
    wb2 = (wb9[None, :, :, None] * eyeb[:, None, None, :]).reshape(
        B * Cout, 9 * B)

    kern = functools.partial(_fused_kernel, W=W, Wa=Wa, HW=HW, Ma=Ma, B=B,
                             Cout=Cout)
    out = pl.pallas_call(
        kern,
        out_shape=jax.ShapeDtypeStruct((N, Cout, HW), dtype),
        grid=(N // B,),
        in_specs=[
            pl.BlockSpec((B, Cin, HW), lambda n: (n, 0, 0)),
            pl.BlockSpec((Cout + 1, Cin * 9), lambda n: (0, 0)),
            pl.BlockSpec((HW, Ma), lambda n: (0, 0)),
            pl.BlockSpec((B * Cout, 9 * B), lambda n: (0, 0)),
            pl.BlockSpec((Ma, HW), lambda n: (0, 0)),
            pl.BlockSpec((Cout, 1), lambda n: (0, 0)),
            pl.BlockSpec((1, 1), lambda n: (0, 0)),
            pl.BlockSpec((9, 1, HW), lambda n: (0, 0, 0)),
            pl.BlockSpec((9, 1, Ma), lambda n: (0, 0, 0)),
        ],
        out_specs=pl.BlockSpec((B, Cout, HW), lambda n: (n, 0, 0)),
        compiler_params=pltpu.CompilerParams(dimension_semantics=("parallel",)),
        cost_estimate=pl.CostEstimate(flops=flops, transcendentals=0,
                                      bytes_accessed=bytes_accessed),
    )(xf, wc, ssel, wb2, u2, bias, ba, m9, am9)

    return out.reshape(N, Cout, H, W)


# speedup vs baseline: 3.7481x; 1.0021x over previous
import functools

import jax
import jax.numpy as jnp
from jax.experimental import pallas as pl
from jax.experimental.pallas import tpu as pltpu


def _fused_kernel(x_ref, wc_ref, ssel_ref, wb2_ref, u2_ref, bias_ref, ba_ref,
                  m_ref, am_ref, o_ref, *, W, Wa, HW, Ma, B, Cout):
    parts_all = []
    for b in range(B):
        xv = x_ref[b]
        parts = []
        for t in range(9):
            kh, kw = divmod(t, 3)
            off = (kh - 1) * W + (kw - 1)
            r = pltpu.roll(xv, (-off) % HW, axis=1) if off != 0 else xv
            if t != 4:
                r = r * m_ref[t]
            parts.append(r)
        parts_all.append(jnp.concatenate(parts, axis=0))
    p_all = jnp.concatenate(parts_all, axis=1)

    acc9 = jnp.dot(wc_ref[...], p_all, preferred_element_type=jnp.float32)

    v_all = jnp.concatenate(
        [acc9[Cout:Cout + 1, b * HW:(b + 1) * HW] for b in range(B)],
        axis=0)
    a_all = jnp.dot(v_all, ssel_ref[...],
                    preferred_element_type=jnp.float32) + ba_ref[...]

    aparts = []
    for t in range(9):
        kh, kw = divmod(t, 3)
        off = (kh - 1) * Wa + (kw - 1)
        r = pltpu.roll(a_all, (-off) % Ma, axis=1) if off != 0 else a_all
        if t != 4:
            r = r * am_ref[t]
        aparts.append(r)
    a9 = jnp.concatenate(aparts, axis=0)

    ls = jnp.dot(wb2_ref[...], a9, preferred_element_type=jnp.float32)
    up = jnp.dot(ls, u2_ref[...],
                 preferred_element_type=jnp.float32)

    for b in range(B):
        o_ref[b] = (acc9[:Cout, b * HW:(b + 1) * HW]
                    + up[b * Cout:(b + 1) * Cout]
                    + bias_ref[...]).astype(o_ref.dtype)


def kernel(x, w_fixed, b_fixed, w_a, b_a, w_b, b_b):
    N, Cin, H, W = x.shape
    Cout = w_fixed.shape[0]
    HW = H * W
    Ha, Wa = H // 4, W // 4
    Ma = Ha * Wa
    dtype = x.dtype
    bf = jnp.bfloat16

    xf = x.reshape(N, Cin, HW)

    wc = jnp.concatenate([
        jnp.transpose(w_fixed, (0, 2, 3, 1)).reshape(Cout, Cin * 9),
        jnp.transpose(w_a, (0, 2, 3, 1)).reshape(1, Cin * 9),
    ], axis=0)
    wb9 = w_b.reshape(Cout, 9)
    bias = (b_fixed + b_b).reshape(Cout, 1)
    ba = b_a.reshape(1, 1)

    hh = jnp.arange(HW) // W
    ww = jnp.arange(HW) % W
    ha = jnp.arange(Ma) // Wa
    wa_ = jnp.arange(Ma) % Wa
    masks, amasks = [], []
    for t in range(9):
        kh, kw = divmod(t, 3)
        masks.append(((hh + kh - 1 >= 0) & (hh + kh - 1 < H)
                      & (ww + kw - 1 >= 0) & (ww + kw - 1 < W)))
        amasks.append(((ha + kh - 1 >= 0) & (ha + kh - 1 < Ha)
                       & (wa_ + kw - 1 >= 0) & (wa_ + kw - 1 < Wa)))
    m9 = jnp.stack(masks).reshape(9, 1, HW).astype(jnp.float32)
    am9 = jnp.stack(amasks).reshape(9, 1, Ma).astype(jnp.float32)

    q_of_m = (ha * 4) * W + wa_ * 4
    ssel = (jnp.arange(HW)[:, None] == q_of_m[None, :]).astype(jnp.float32)
    m_of_q = (hh // 4) * Wa + ww // 4
    u2 = (jnp.arange(Ma)[:, None] == m_of_q[None, :]).astype(jnp.float32)

    flops = int(N * (2 * (Cout + 1) * Cin * 9 * HW + 2 * HW * Ma
                     + 2 * Cout * 9 * Ma + 2 * Cout * Ma * HW))
    bytes_accessed = int(4 * (N * Cin * HW + N * Cout * HW)
                         + 2 * (HW * Ma * 2 + 9 * HW + 9 * Ma))

    B = 32
    while N % B:
        B //= 2

    eyeb = jnp.eye(B, dtype=jnp.float32)
    wb2 = (wb9[None, :, :, None] * eyeb[:, None, None, :]).reshape(
        B * Cout, 9 * B)

    kern = functools.partial(_fused_kernel, W=W, Wa=Wa, HW=HW, Ma=Ma, B=B,
                             Cout=Cout)
    out = pl.pallas_call(
        kern,
        out_shape=jax.ShapeDtypeStruct((N, Cout, HW), dtype),
        grid=(N // B,),
        in_specs=[
            pl.BlockSpec((B, Cin, HW), lambda n: (n, 0, 0)),
            pl.BlockSpec((Cout + 1, Cin * 9), lambda n: (0, 0)),
            pl.BlockSpec((HW, Ma), lambda n: (0, 0)),
            pl.BlockSpec((B * Cout, 9 * B), lambda n: (0, 0)),
            pl.BlockSpec((Ma, HW), lambda n: (0, 0)),
            pl.BlockSpec((Cout, 1), lambda n: (0, 0)),
            pl.BlockSpec((1, 1), lambda n: (0, 0)),
            pl.BlockSpec((9, 1, HW), lambda n: (0, 0, 0)),
            pl.BlockSpec((9, 1, Ma), lambda n: (0, 0, 0)),
        ],
        out_specs=pl.BlockSpec((B, Cout, HW), lambda n: (n, 0, 0)),
        compiler_params=pltpu.CompilerParams(dimension_semantics=("parallel",)),
        cost_estimate=pl.CostEstimate(flops=flops, transcendentals=0,
                                      bytes_accessed=bytes_accessed),
    )(xf, wc, ssel, wb2, u2, bias, ba, m9, am9)

    return out.reshape(N, Cout, H, W)
